# baseline (speedup 1.0000x reference)
import functools
import itertools

import numpy as np
import jax
import jax.numpy as jnp
from jax import lax
from jax.experimental import pallas as pl
from jax.experimental.pallas import tpu as pltpu
from jax.experimental.pallas import tpu_sc as plsc

N = 10000
E = 320000
BG = 64
HID = 64
CW = 16
GW = 128
SW = 128
EB = 2048
CH = 128
NWK = 32
EPAD = 327680
GR = 2 * EPAD
NP_ = 10112
NBLK = 2000
EP_SMALL = 2048
BNS = float(1.0 / np.sqrt(1.0 + 1e-5))
F32 = jnp.float32

_INTERP = False


def _silu(x):
    return x * jax.nn.sigmoid(x)


def _mesh():
    return plsc.VectorSubcoreMesh(core_axis_name="c", subcore_axis_name="s")


def _rows_call(fn, n_rows, blk, rows, consts, out_widths, out_dtypes=None):
    grid = (n_rows // blk,)
    if out_dtypes is None:
        out_dtypes = [F32] * len(out_widths)
    in_specs = (
        [pl.BlockSpec((blk, r.shape[1]), lambda i: (i, 0)) for r in rows]
        + [pl.BlockSpec(c.shape, lambda i, nd=c.ndim: (0,) * nd) for c in consts]
    )
    out_specs = [pl.BlockSpec((blk, w), lambda i: (i, 0)) for w in out_widths]
    out_shape = [jax.ShapeDtypeStruct((n_rows, w), d)
                 for w, d in zip(out_widths, out_dtypes)]
    nr, nc = len(rows), len(consts)

    def body(*refs):
        fn(refs[:nr], refs[nr:nr + nc], refs[nr + nc:], pl.program_id(0))

    return pl.pallas_call(body, grid=grid, in_specs=in_specs,
                          out_specs=out_specs, out_shape=out_shape,
                          interpret=_INTERP)(*rows, *consts)


def _acc_call(fn, n_rows, blk, rows, consts, out_shapes):
    grid = (n_rows // blk,)
    in_specs = (
        [pl.BlockSpec((blk, r.shape[1]), lambda i: (i, 0)) for r in rows]
        + [pl.BlockSpec(c.shape, lambda i, nd=c.ndim: (0,) * nd) for c in consts]
    )
    out_specs = [pl.BlockSpec(s, lambda i, nd=len(s): (0,) * nd)
                 for s in out_shapes]
    out_shape = [jax.ShapeDtypeStruct(s, F32) for s in out_shapes]
    nr, nc = len(rows), len(consts)

    def body(*refs):
        fn(refs[:nr], refs[nr:nr + nc], refs[nr + nc:], pl.program_id(0))

    return pl.pallas_call(body, grid=grid, in_specs=in_specs,
                          out_specs=out_specs, out_shape=out_shape,
                          interpret=_INTERP)(*rows, *consts)


def _full_call(fn, ins, out_shapes):
    in_specs = [pl.BlockSpec(a.shape, lambda nd=a.ndim: (0,) * nd) for a in ins]
    out_specs = [pl.BlockSpec(s, lambda nd=len(s): (0,) * nd)
                 for s in out_shapes]
    out_shape = [jax.ShapeDtypeStruct(s, F32) for s in out_shapes]
    ni = len(ins)

    def body(*refs):
        fn(refs[:ni], refs[ni:])

    return pl.pallas_call(body, grid=(), in_specs=in_specs,
                          out_specs=out_specs, out_shape=out_shape,
                          interpret=_INTERP)(*ins)


def _mm_fn(ins, consts, outs, i):
    x, = ins
    W, b = (c[...] for c in consts)
    outs[0][...] = jnp.dot(x[...], W, preferred_element_type=F32) + b


def _tables_call(h, coord, W1, b1):
    def body(hr, cdr, W1r, b1r, out):
        k = pl.program_id(0)
        h_ = hr[...]
        cd = cdr[...]
        W = W1r[...]
        chi = cd.astype(jnp.bfloat16)
        clo = (cd - chi.astype(F32)).astype(jnp.bfloat16)
        z = jnp.zeros((h_.shape[0], GW - HID - 2 * CW), jnp.bfloat16)

        @pl.when(k == 0)
        def _():
            A = jnp.dot(h_, W[0:HID], preferred_element_type=F32) + b1r[...]
            out[0] = jnp.concatenate([A.astype(jnp.bfloat16), chi, clo, z], 1)

        @pl.when(k == 1)
        def _():
            Bv = jnp.dot(h_, W[HID:2 * HID], preferred_element_type=F32)
            out[0] = jnp.concatenate([Bv.astype(jnp.bfloat16), chi, clo, z], 1)

    return pl.pallas_call(
        body, grid=(2, N // NBLK),
        in_specs=[pl.BlockSpec((NBLK, HID), lambda k, i: (i, 0)),
                  pl.BlockSpec((NBLK, CW), lambda k, i: (i, 0)),
                  pl.BlockSpec(W1.shape, lambda k, i: (0, 0)),
                  pl.BlockSpec(b1.shape, lambda k, i: (0, 0))],
        out_specs=pl.BlockSpec((1, NBLK, GW), lambda k, i: (k, i, 0)),
        out_shape=jax.ShapeDtypeStruct((2, N, GW), jnp.bfloat16),
        interpret=_INTERP)(h, coord, W1, b1)


def _edge_call(gath, ea, prm):
    grid = (EPAD // EB,)
    consts = [prm["W2"], prm["b2"], prm["Watt"], prm["batt"], prm["Wc1"],
              prm["bc1"], prm["Wc2"], prm["W1"]]

    def body(g1r, g2r, ear, W2, b2, Watt, batt, Wc1, bc1, Wc2, W1r, out):
        g1 = g1r[...]
        g2 = g2r[...]
        wr = W1r[2 * HID:2 * HID + 1]
        we = W1r[2 * HID + 1:2 * HID + 2]
        a = g1[:, :HID].astype(F32) + g2[:, :HID].astype(F32)
        c1 = (g1[:, HID:HID + CW].astype(F32)
              + g1[:, HID + CW:HID + 2 * CW].astype(F32))
        c2 = (g2[:, HID:HID + CW].astype(F32)
              + g2[:, HID + CW:HID + 2 * CW].astype(F32))
        cdd = c1 - c2
        radial = jnp.sum(cdd * cdd, axis=1, keepdims=True)
        e1 = _silu(a + radial * wr + ear[...] * we)
        m = _silu(jnp.dot(e1, W2[...], preferred_element_type=F32) + b2[...])
        att = jax.nn.sigmoid(
            jnp.dot(m, Watt[...], preferred_element_type=F32) + batt[...])
        m = m * att
        t = jnp.tanh(jnp.dot(
            _silu(jnp.dot(m, Wc1[...], preferred_element_type=F32) + bc1[...]),
            Wc2[...], preferred_element_type=F32))
        cdt = cdd * (t / (jnp.sqrt(radial) + 1e-8))
        lane = lax.broadcasted_iota(jnp.int32, (EB, CW), 1)
        cdt = jnp.where(lane == CW - 1, 1.0, cdt)
        z = jnp.zeros((EB, SW - HID - CW), F32)
        out[...] = jnp.concatenate([m, cdt, z], axis=1)

    in_specs = (
        [pl.BlockSpec((EB, GW), lambda i: (i, 0)),
         pl.BlockSpec((EB, GW), lambda i: (i + EPAD // EB, 0)),
         pl.BlockSpec((EB, 1), lambda i: (i, 0))]
        + [pl.BlockSpec(c.shape, lambda i, nd=c.ndim: (0,) * nd)
           for c in consts]
    )
    return pl.pallas_call(
        body, grid=grid, in_specs=in_specs,
        out_specs=pl.BlockSpec((EB, SW), lambda i: (i, 0)),
        out_shape=jax.ShapeDtypeStruct((EPAD, SW), F32),
        interpret=_INTERP)(gath, gath, ea, *consts)


def _update_fn(ins, consts, outs, i):
    h, cd, p0, p1 = (r[...] for r in ins)
    Wn1, bn1, Wn2, bn2 = (c[...] for c in consts)
    P = p0 + p1
    agg = P[:, :HID]
    u = _silu(jnp.dot(h, Wn1[0:HID], preferred_element_type=F32)
              + jnp.dot(agg, Wn1[HID:2 * HID], preferred_element_type=F32)
              + bn1)
    outs[0][...] = h + jnp.dot(u, Wn2, preferred_element_type=F32) + bn2
    rest = P[:, HID:HID + CW]
    cnt = jnp.maximum(rest[:, CW - 1:CW], 1.0)
    lane = lax.broadcasted_iota(jnp.int32, rest.shape, 1)
    outs[1][...] = cd + jnp.where(lane < 3, rest, 0.0) / cnt


def _trans_fn(ins, consts, outs, i):
    h, = ins
    Wo, bo = consts[0][...], consts[1][...]
    o = jnp.dot(h[...], Wo, preferred_element_type=F32) + bo
    outs[0][...] = o
    if len(outs) > 1:
        Wi, bi = consts[2][...], consts[3][...]
        outs[1][...] = jnp.dot(o, Wi, preferred_element_type=F32) + bi


def _pool_fn(ins, consts, outs, i):
    o1, o2, o3, cd, xb = (r[...] for r in ins)
    oh = (xb == lax.broadcasted_iota(jnp.int32, (xb.shape[0], BG), 1)
          ).astype(F32)
    lane = lax.broadcasted_iota(jnp.int32, cd.shape, 1)
    cdc = jnp.where(lane < 3, cd, 0.0)
    cdc = jnp.where(lane == CW - 1, 1.0, cdc)
    dn = (((0,), (0,)), ((), ()))
    for o_ref, dat in zip(outs, (o1, o2, o3, cdc)):
        v = lax.dot_general(oh, dat, dn, preferred_element_type=F32)

        @pl.when(i == 0)
        def _():
            o_ref[...] = v

        @pl.when(i > 0)
        def _():
            o_ref[...] = o_ref[...] + v


def _small_body(ins, outs):
    (h0r, csr, Rr, Cr, Winr, binr, W1a, W1b, W2, Wc1, Wn1a, Wn1b, Wn2, WV,
     AUX, Woutr, boutr) = ins
    o_ref, co_ref = outs
    Rg = Rr[...]
    Cg = Cr[...]
    lane = lax.broadcasted_iota(jnp.int32, (BG, CW), 1)
    elane = lax.broadcasted_iota(jnp.int32, (EP_SMALL, CW), 1)
    cs = csr[...]
    cnt0 = jnp.maximum(cs[:, CW - 1:CW], 1.0)
    coord = jnp.where(lane < 3, cs / cnt0, 0.0)
    h = jnp.dot(h0r[...], Winr[...], preferred_element_type=F32) + binr[...]
    dn = (((0,), (0,)), ((), ()))
    for l in range(2):
        aux = AUX[l]
        wr, b1, b2, bc1, bn1, bn2 = (aux[k:k + 1] for k in range(6))
        batt = aux[6:7, 0:1]
        wv = WV[l]
        hr = jnp.dot(Rg, h, preferred_element_type=F32)
        hc = jnp.dot(Cg, h, preferred_element_type=F32)
        cr = jnp.dot(Rg, coord, preferred_element_type=F32)
        cc = jnp.dot(Cg, coord, preferred_element_type=F32)
        cdd = cr - cc
        radial = jnp.sum(cdd * cdd, axis=1, keepdims=True)
        e1 = _silu(jnp.dot(hr, W1a[l], preferred_element_type=F32)
                   + jnp.dot(hc, W1b[l], preferred_element_type=F32)
                   + radial * wr + b1)
        m = _silu(jnp.dot(e1, W2[l], preferred_element_type=F32) + b2)
        att = jax.nn.sigmoid(
            jnp.dot(m, wv[:, 0:1], preferred_element_type=F32) + batt)
        m = m * att
        t = jnp.tanh(jnp.dot(
            _silu(jnp.dot(m, Wc1[l], preferred_element_type=F32) + bc1),
            wv[:, 1:2], preferred_element_type=F32))
        cdt = cdd * (t / (jnp.sqrt(radial) + 1e-8))
        cdt = jnp.where(elane == CW - 1, 1.0, cdt)
        agg = lax.dot_general(Rg, m, dn, preferred_element_type=F32)
        csum = lax.dot_general(Rg, cdt, dn, preferred_element_type=F32)
        cnt = jnp.maximum(csum[:, CW - 1:CW], 1.0)
        coord = coord + jnp.where(lane < 3, csum, 0.0) / cnt
        u = _silu(jnp.dot(h, Wn1a[l], preferred_element_type=F32)
                  + jnp.dot(agg, Wn1b[l], preferred_element_type=F32) + bn1)
        h = h + jnp.dot(u, Wn2[l], preferred_element_type=F32) + bn2
    o_ref[...] = jnp.dot(h, Woutr[...], preferred_element_type=F32) + boutr[...]
    co_ref[...] = jnp.where(lane == CW - 1, 1.0, coord)


def _head_body(ins, outs):
    (p1, p2, p3, pc, o4, o5, o6, g1, be1, g2, be2, g3, be3,
     Wfc, bfc, gfc, befc, Wfn, bfn, gfn, befn) = (r[...] for r in ins)
    cnt = jnp.maximum(pc[:, CW - 1:CW], 1.0)

    def bnrelu(x, g, b):
        return jax.nn.relu(x * (g * BNS) + b)

    r1 = bnrelu(p1 / cnt, g1, be1)
    r2 = bnrelu(p2 / cnt, g2, be2)
    r3 = bnrelu(p3 / cnt, g3, be3)
    r4 = bnrelu(o4, g1, be1)
    r5 = bnrelu(o5, g2, be2)
    r6 = bnrelu(o6, g3, be3)
    cat = jnp.concatenate([r1, r4, r2, r5, r3, r6], axis=1)
    x1 = bnrelu(jnp.dot(cat, Wfc, preferred_element_type=F32) + bfc, gfc, befc)
    x2 = bnrelu(jnp.dot(x1, Wfn, preferred_element_type=F32) + bfn, gfn, befn)
    outs[0][...] = jax.nn.sigmoid(x2)


def _sc_gather(tab, gidx):
    nchunks = GR // (NWK * CH)
    cpw = nchunks * CH

    @functools.partial(
        pl.kernel,
        out_type=jax.ShapeDtypeStruct((GR, GW), jnp.bfloat16),
        mesh=_mesh(),
        compiler_params=pltpu.CompilerParams(use_tc_tiling_on_sc=False),
        scratch_types=[pltpu.VMEM((cpw,), jnp.int32),
                       pltpu.VMEM((4, CH, GW), jnp.bfloat16)]
        + [pltpu.SemaphoreType.DMA] * 8,
    )
    def gk(tab_hbm, gidx_hbm, out_hbm, idx_v, rows_v, *sems):
        semg, semw = sems[:4], sems[4:]
        wid = lax.axis_index("s") * 2 + lax.axis_index("c")
        pltpu.sync_copy(gidx_hbm.at[pl.ds(wid * cpw, cpw)], idx_v)

        def start_gather(g, b):
            pltpu.async_copy(
                tab_hbm.at[idx_v.at[pl.ds(g * CH, CH)]], rows_v.at[b],
                semg[b])

        def start_write(g, b):
            pltpu.async_copy(
                rows_v.at[b],
                out_hbm.at[pl.ds(wid * cpw + g * CH, CH), :], semw[b])

        def wait_gather(b):
            pltpu.make_async_copy(
                tab_hbm.at[pl.ds(0, CH), :], rows_v.at[b], semg[b]).wait()

        def wait_write(b):
            pltpu.make_async_copy(
                rows_v.at[b], out_hbm.at[pl.ds(0, CH), :], semw[b]).wait()

        start_gather(0, 0)
        start_gather(1, 1)

        def body(o, carry):
            for b in range(4):
                g = o * 4 + b
                bb = (b + 2) % 4

                @pl.when(jnp.logical_and(g >= 2, g + 2 < nchunks))
                def _():
                    wait_write(bb)

                @pl.when(g + 2 < nchunks)
                def _():
                    start_gather(g + 2, bb)

                wait_gather(b)
                start_write(g, b)
            return carry

        lax.fori_loop(0, nchunks // 4, body, 0)
        for b in range(4):
            wait_write(b)

    return gk(tab, gidx)


def _sc_scatter(src, ridx, zacc):
    nch = EPAD // (NWK * CH)
    rows_pt = NP_ // 16

    @functools.partial(
        pl.kernel,
        out_type=jax.ShapeDtypeStruct((2, NP_, SW), F32),
        mesh=_mesh(),
        compiler_params=pltpu.CompilerParams(use_tc_tiling_on_sc=True),
        scratch_types=[pltpu.VMEM((nch, CH), jnp.int32),
                       pltpu.VMEM((2, CH, SW), F32),
                       pltpu.VMEM_SHARED((NP_, SW), F32)]
        + [pltpu.SemaphoreType.DMA] * 2,
    )
    def sk(src_hbm, ridx_hbm, z_hbm, out_hbm, idx_v, src_v, acc_sh, *sems):
        semr = sems
        c = lax.axis_index("c")
        s = lax.axis_index("s")
        wid = s * 2 + c
        pltpu.sync_copy(ridx_hbm.at[wid], idx_v)
        pltpu.sync_copy(z_hbm, acc_sh.at[pl.ds(s * rows_pt, rows_pt), :])
        plsc.subcore_barrier()

        def start_load(g, b):
            pltpu.async_copy(
                src_hbm.at[pl.ds((wid * nch + g) * CH, CH), :],
                src_v.at[b], semr[b])

        def wait_load(b):
            pltpu.make_async_copy(
                src_hbm.at[pl.ds(0, CH), :], src_v.at[b], semr[b]).wait()

        for b in range(2):
            start_load(b, b)

        def body(o, carry):
            for b in range(2):
                g = o * 2 + b
                wait_load(b)
                pltpu.sync_copy(src_v.at[b], acc_sh.at[idx_v.at[g]],
                                add=True)

                @pl.when(g + 2 < nch)
                def _():
                    start_load(g + 2, b)
            return carry

        lax.fori_loop(0, nch // 2, body, 0)
        plsc.subcore_barrier()
        pltpu.sync_copy(acc_sh.at[pl.ds(s * rows_pt, rows_pt), :],
                        out_hbm.at[c, pl.ds(s * rows_pt, rows_pt), :])

    return sk(src, ridx, zacc)


def _b(v):
    return v.reshape(1, -1)


def _prep_gcl(p):
    return {
        "W1": p["edge1"]["W"],
        "b1": _b(p["edge1"]["b"]),
        "W2": p["edge2"]["W"], "b2": _b(p["edge2"]["b"]),
        "Watt": p["att"]["W"], "batt": _b(p["att"]["b"]),
        "Wc1": p["coord1"]["W"], "bc1": _b(p["coord1"]["b"]),
        "Wc2": p["coord2W"],
        "Wn1": p["node1"]["W"], "bn1": _b(p["node1"]["b"]),
        "Wn2": p["node2"]["W"], "bn2": _b(p["node2"]["b"]),
    }


def _small_onehots():
    pairs = np.array(list(itertools.combinations(range(BG), 2)),
                     dtype=np.int64).T
    R = np.zeros((EP_SMALL, BG), np.float32)
    C = np.zeros((EP_SMALL, BG), np.float32)
    ne = pairs.shape[1]
    R[np.arange(ne), pairs[0]] = 1.0
    C[np.arange(ne), pairs[1]] = 1.0
    return jnp.asarray(R), jnp.asarray(C)


def _small_params(gp):
    gcls = gp["gcls"]
    W1s, W1bs, W2s, Wc1s, Wn1as, Wn1bs, Wn2s, WVs, AUXs = \
        [], [], [], [], [], [], [], [], []
    for p in gcls:
        W1 = p["edge1"]["W"]
        W1s.append(W1[0:HID])
        W1bs.append(W1[HID:2 * HID])
        W2s.append(p["edge2"]["W"])
        Wc1s.append(p["coord1"]["W"])
        Wn1as.append(p["node1"]["W"][0:HID])
        Wn1bs.append(p["node1"]["W"][HID:2 * HID])
        Wn2s.append(p["node2"]["W"])
        WVs.append(jnp.concatenate([p["att"]["W"], p["coord2W"]], axis=1))
        aux = jnp.stack([
            W1[2 * HID], p["edge1"]["b"], p["edge2"]["b"], p["coord1"]["b"],
            p["node1"]["b"], p["node2"]["b"],
            jnp.full((HID,), p["att"]["b"][0], F32), jnp.zeros((HID,), F32)])
        AUXs.append(aux)
    st = lambda xs: jnp.stack(xs)
    return (st(W1s), st(W1bs), st(W2s), st(Wc1s), st(Wn1as), st(Wn1bs),
            st(Wn2s), st(WVs), st(AUXs),
            gp["emb_in"]["W"], _b(gp["emb_in"]["b"]),
            gp["emb_out"]["W"], _b(gp["emb_out"]["b"]))


def kernel(x_res, x_emb_seq, edge_attr, x_pos, params, edge_index, x_batch):
    row = edge_index[0].astype(jnp.int32)
    col = edge_index[1].astype(jnp.int32)
    npadr = (EPAD - E) // CH
    row2 = row.reshape(E // CH, CH)
    colN2 = (col + N).reshape(E // CH, CH)
    zpad = jnp.zeros((npadr, CH), jnp.int32)
    gidx = jnp.concatenate([row2, zpad, colN2, zpad], 0).reshape(GR)
    ridx = jnp.concatenate(
        [row2, jnp.full((npadr, CH), N, jnp.int32)],
        0).reshape(NWK, EPAD // (NWK * CH), CH)
    ea = jnp.concatenate(
        [edge_attr[:, 0:1].reshape(E // CH, CH),
         jnp.zeros((npadr, CH), F32)], 0).reshape(EPAD, 1)
    coord0 = jnp.pad(x_pos, ((0, 0), (0, CW - 3)))
    zacc = jnp.zeros((NP_ // 16, SW), F32)
    xb = x_batch.astype(jnp.int32).reshape(N, 1)

    p1 = params["egnn1"]
    h = _rows_call(_mm_fn, N, NBLK, [x_res],
                   [p1["emb_in"]["W"], _b(p1["emb_in"]["b"])], [HID])[0]
    coord = coord0
    os_ = []
    for e in (1, 2, 3):
        gp = params[f"egnn{e}"]
        for l in range(2):
            prm = _prep_gcl(gp["gcls"][l])
            tab = _tables_call(h, coord, prm["W1"],
                               prm["b1"]).reshape(2 * N, GW)
            gath = _sc_gather(tab, gidx)
            src = _edge_call(gath, ea, prm)
            parts = _sc_scatter(src, ridx, zacc)
            h, coord = _rows_call(
                _update_fn, N, NBLK,
                [h, coord, parts[0, :N], parts[1, :N]],
                [prm["Wn1"], prm["bn1"], prm["Wn2"], prm["bn2"]],
                [HID, CW])
        consts = [gp["emb_out"]["W"], _b(gp["emb_out"]["b"])]
        widths = [gp["emb_out"]["W"].shape[1]]
        if e < 3:
            nxt = params[f"egnn{e + 1}"]
            consts += [nxt["emb_in"]["W"], _b(nxt["emb_in"]["b"])]
            widths += [HID]
            o, h = _rows_call(_trans_fn, N, NBLK, [h], consts, widths)
        else:
            o, = _rows_call(_trans_fn, N, NBLK, [h], consts, widths)
        os_.append(o)

    pool1, pool2, pool3, poolc = _acc_call(
        _pool_fn, N, NBLK, [os_[0], os_[1], os_[2], coord0, xb], [],
        [(BG, os_[0].shape[1]), (BG, os_[1].shape[1]),
         (BG, os_[2].shape[1]), (BG, CW)])

    Rg, Cg = _small_onehots()
    hs, cs = x_emb_seq, poolc
    small_os = []
    for e in (4, 5, 6):
        (W1a, W1b, W2, Wc1, Wn1a, Wn1b, Wn2, WV, AUX,
         Win, bin_, Wout, bout) = _small_params(params[f"egnn{e}"])
        o, cs = _full_call(
            _small_body,
            [hs, cs, Rg, Cg, Win, bin_, W1a, W1b, W2, Wc1, Wn1a, Wn1b,
             Wn2, WV, AUX, Wout, bout],
            [(BG, Wout.shape[1]), (BG, CW)])
        small_os.append(o)
        hs = o

    fp = params["fc1"]
    fn = params["final"]
    out, = _full_call(
        _head_body,
        [pool1, pool2, pool3, poolc, small_os[0], small_os[1], small_os[2],
         _b(params["bnrelu1"]["gamma"]), _b(params["bnrelu1"]["beta"]),
         _b(params["bnrelu2"]["gamma"]), _b(params["bnrelu2"]["beta"]),
         _b(params["bnrelu3"]["gamma"]), _b(params["bnrelu3"]["beta"]),
         fp["lin"]["W"], _b(fp["lin"]["b"]), _b(fp["gamma"]), _b(fp["beta"]),
         fn["lin"]["W"], _b(fn["lin"]["b"]), _b(fn["gamma"]), _b(fn["beta"])],
        [(BG, fn["lin"]["W"].shape[1])])
    return out


# speedup vs baseline: 1.1786x; 1.1786x over previous
import functools
import itertools

import numpy as np
import jax
import jax.numpy as jnp
from jax import lax
from jax.experimental import pallas as pl
from jax.experimental.pallas import tpu as pltpu
from jax.experimental.pallas import tpu_sc as plsc

N = 10000
E = 320000
BG = 64
HID = 64
CW = 16
GW = 128
SW = 128
EB = 2048
CH = 128
NWK = 32
EPAD = 327680
GR = 2 * EPAD
NP_ = 10112
NBLK = 2000
EP_SMALL = 2048
BNS = float(1.0 / np.sqrt(1.0 + 1e-5))
F32 = jnp.float32

_INTERP = False


def _silu(x):
    return x * jax.nn.sigmoid(x)


def _mesh():
    return plsc.VectorSubcoreMesh(core_axis_name="c", subcore_axis_name="s")


def _rows_call(fn, n_rows, blk, rows, consts, out_widths, out_dtypes=None):
    grid = (n_rows // blk,)
    if out_dtypes is None:
        out_dtypes = [F32] * len(out_widths)
    in_specs = (
        [pl.BlockSpec((blk, r.shape[1]), lambda i: (i, 0)) for r in rows]
        + [pl.BlockSpec(c.shape, lambda i, nd=c.ndim: (0,) * nd) for c in consts]
    )
    out_specs = [pl.BlockSpec((blk, w), lambda i: (i, 0)) for w in out_widths]
    out_shape = [jax.ShapeDtypeStruct((n_rows, w), d)
                 for w, d in zip(out_widths, out_dtypes)]
    nr, nc = len(rows), len(consts)

    def body(*refs):
        fn(refs[:nr], refs[nr:nr + nc], refs[nr + nc:], pl.program_id(0))

    return pl.pallas_call(body, grid=grid, in_specs=in_specs,
                          out_specs=out_specs, out_shape=out_shape,
                          interpret=_INTERP)(*rows, *consts)


def _acc_call(fn, n_rows, blk, rows, consts, out_shapes):
    grid = (n_rows // blk,)
    in_specs = (
        [pl.BlockSpec((blk, r.shape[1]), lambda i: (i, 0)) for r in rows]
        + [pl.BlockSpec(c.shape, lambda i, nd=c.ndim: (0,) * nd) for c in consts]
    )
    out_specs = [pl.BlockSpec(s, lambda i, nd=len(s): (0,) * nd)
                 for s in out_shapes]
    out_shape = [jax.ShapeDtypeStruct(s, F32) for s in out_shapes]
    nr, nc = len(rows), len(consts)

    def body(*refs):
        fn(refs[:nr], refs[nr:nr + nc], refs[nr + nc:], pl.program_id(0))

    return pl.pallas_call(body, grid=grid, in_specs=in_specs,
                          out_specs=out_specs, out_shape=out_shape,
                          interpret=_INTERP)(*rows, *consts)


def _full_call(fn, ins, out_shapes):
    in_specs = [pl.BlockSpec(a.shape, lambda nd=a.ndim: (0,) * nd) for a in ins]
    out_specs = [pl.BlockSpec(s, lambda nd=len(s): (0,) * nd)
                 for s in out_shapes]
    out_shape = [jax.ShapeDtypeStruct(s, F32) for s in out_shapes]
    ni = len(ins)

    def body(*refs):
        fn(refs[:ni], refs[ni:])

    return pl.pallas_call(body, grid=(), in_specs=in_specs,
                          out_specs=out_specs, out_shape=out_shape,
                          interpret=_INTERP)(*ins)


def _mm_fn(ins, consts, outs, i):
    x, = ins
    W, b = (c[...] for c in consts)
    outs[0][...] = jnp.dot(x[...], W, preferred_element_type=F32) + b


def _tables_call(h, coord, W1, b1):
    def body(hr, cdr, W1r, b1r, out):
        k = pl.program_id(0)
        h_ = hr[...]
        cd = cdr[...]
        W = W1r[...]
        z = jnp.zeros((h_.shape[0], GW - HID - CW), F32)

        @pl.when(k == 0)
        def _():
            A = jnp.dot(h_, W[0:HID], preferred_element_type=F32) + b1r[...]
            out[0] = jnp.concatenate([A, cd, z], 1)

        @pl.when(k == 1)
        def _():
            Bv = jnp.dot(h_, W[HID:2 * HID], preferred_element_type=F32)
            out[0] = jnp.concatenate([Bv, -cd, z], 1)

    return pl.pallas_call(
        body, grid=(2, N // NBLK),
        in_specs=[pl.BlockSpec((NBLK, HID), lambda k, i: (i, 0)),
                  pl.BlockSpec((NBLK, CW), lambda k, i: (i, 0)),
                  pl.BlockSpec(W1.shape, lambda k, i: (0, 0)),
                  pl.BlockSpec(b1.shape, lambda k, i: (0, 0))],
        out_specs=pl.BlockSpec((1, NBLK, GW), lambda k, i: (k, i, 0)),
        out_shape=jax.ShapeDtypeStruct((2, N, GW), F32),
        interpret=_INTERP)(h, coord, W1, b1)


def _edge_call(gath, ea, prm):
    grid = (EPAD // EB,)
    consts = [prm["W2"], prm["b2"], prm["Watt"], prm["batt"], prm["Wc1"],
              prm["bc1"], prm["Wc2"], prm["W1"]]

    def body(g1r, ear, W2, b2, Watt, batt, Wc1, bc1, Wc2, W1r, out):
        g1 = g1r[...]
        wr = W1r[2 * HID:2 * HID + 1]
        we = W1r[2 * HID + 1:2 * HID + 2]
        a = g1[:, :HID]
        cdd = g1[:, HID:HID + CW]
        radial = jnp.sum(cdd * cdd, axis=1, keepdims=True)
        e1 = _silu(a + radial * wr + ear[...] * we)
        m = _silu(jnp.dot(e1, W2[...], preferred_element_type=F32) + b2[...])
        att = jax.nn.sigmoid(
            jnp.dot(m, Watt[...], preferred_element_type=F32) + batt[...])
        m = m * att
        t = jnp.tanh(jnp.dot(
            _silu(jnp.dot(m, Wc1[...], preferred_element_type=F32) + bc1[...]),
            Wc2[...], preferred_element_type=F32))
        cdt = cdd * (t / (jnp.sqrt(radial) + 1e-8))
        lane = lax.broadcasted_iota(jnp.int32, (EB, CW), 1)
        cdt = jnp.where(lane == CW - 1, 1.0, cdt)
        z = jnp.zeros((EB, SW - HID - CW), F32)
        out[...] = jnp.concatenate([m, cdt, z], axis=1)

    in_specs = (
        [pl.BlockSpec((EB, GW), lambda i: (i, 0)),
         pl.BlockSpec((EB, 1), lambda i: (i, 0))]
        + [pl.BlockSpec(c.shape, lambda i, nd=c.ndim: (0,) * nd)
           for c in consts]
    )
    return pl.pallas_call(
        body, grid=grid, in_specs=in_specs,
        out_specs=pl.BlockSpec((EB, SW), lambda i: (i, 0)),
        out_shape=jax.ShapeDtypeStruct((EPAD, SW), F32),
        interpret=_INTERP)(gath, ea, *consts)


def _update_fn(ins, consts, outs, i):
    h, cd, p0, p1 = (r[...] for r in ins)
    Wn1, bn1, Wn2, bn2 = (c[...] for c in consts)
    P = p0 + p1
    agg = P[:, :HID]
    u = _silu(jnp.dot(h, Wn1[0:HID], preferred_element_type=F32)
              + jnp.dot(agg, Wn1[HID:2 * HID], preferred_element_type=F32)
              + bn1)
    outs[0][...] = h + jnp.dot(u, Wn2, preferred_element_type=F32) + bn2
    rest = P[:, HID:HID + CW]
    cnt = jnp.maximum(rest[:, CW - 1:CW], 1.0)
    lane = lax.broadcasted_iota(jnp.int32, rest.shape, 1)
    outs[1][...] = cd + jnp.where(lane < 3, rest, 0.0) / cnt


def _trans_fn(ins, consts, outs, i):
    h, = ins
    Wo, bo = consts[0][...], consts[1][...]
    o = jnp.dot(h[...], Wo, preferred_element_type=F32) + bo
    outs[0][...] = o
    if len(outs) > 1:
        Wi, bi = consts[2][...], consts[3][...]
        outs[1][...] = jnp.dot(o, Wi, preferred_element_type=F32) + bi


def _pool_fn(ins, consts, outs, i):
    o1, o2, o3, cd, xb = (r[...] for r in ins)
    oh = (xb == lax.broadcasted_iota(jnp.int32, (xb.shape[0], BG), 1)
          ).astype(F32)
    lane = lax.broadcasted_iota(jnp.int32, cd.shape, 1)
    cdc = jnp.where(lane < 3, cd, 0.0)
    cdc = jnp.where(lane == CW - 1, 1.0, cdc)
    dn = (((0,), (0,)), ((), ()))
    for o_ref, dat in zip(outs, (o1, o2, o3, cdc)):
        v = lax.dot_general(oh, dat, dn, preferred_element_type=F32)

        @pl.when(i == 0)
        def _():
            o_ref[...] = v

        @pl.when(i > 0)
        def _():
            o_ref[...] = o_ref[...] + v


def _small_body(ins, outs):
    (h0r, csr, Rr, Cr, Winr, binr, W1a, W1b, W2, Wc1, Wn1a, Wn1b, Wn2, WV,
     AUX, Woutr, boutr) = ins
    o_ref, co_ref = outs
    Rg = Rr[...]
    Cg = Cr[...]
    lane = lax.broadcasted_iota(jnp.int32, (BG, CW), 1)
    elane = lax.broadcasted_iota(jnp.int32, (EP_SMALL, CW), 1)
    cs = csr[...]
    cnt0 = jnp.maximum(cs[:, CW - 1:CW], 1.0)
    coord = jnp.where(lane < 3, cs / cnt0, 0.0)
    h = jnp.dot(h0r[...], Winr[...], preferred_element_type=F32) + binr[...]
    dn = (((0,), (0,)), ((), ()))
    for l in range(2):
        aux = AUX[l]
        wr, b1, b2, bc1, bn1, bn2 = (aux[k:k + 1] for k in range(6))
        batt = aux[6:7, 0:1]
        wv = WV[l]
        hr = jnp.dot(Rg, h, preferred_element_type=F32)
        hc = jnp.dot(Cg, h, preferred_element_type=F32)
        cr = jnp.dot(Rg, coord, preferred_element_type=F32)
        cc = jnp.dot(Cg, coord, preferred_element_type=F32)
        cdd = cr - cc
        radial = jnp.sum(cdd * cdd, axis=1, keepdims=True)
        e1 = _silu(jnp.dot(hr, W1a[l], preferred_element_type=F32)
                   + jnp.dot(hc, W1b[l], preferred_element_type=F32)
                   + radial * wr + b1)
        m = _silu(jnp.dot(e1, W2[l], preferred_element_type=F32) + b2)
        att = jax.nn.sigmoid(
            jnp.dot(m, wv[:, 0:1], preferred_element_type=F32) + batt)
        m = m * att
        t = jnp.tanh(jnp.dot(
            _silu(jnp.dot(m, Wc1[l], preferred_element_type=F32) + bc1),
            wv[:, 1:2], preferred_element_type=F32))
        cdt = cdd * (t / (jnp.sqrt(radial) + 1e-8))
        cdt = jnp.where(elane == CW - 1, 1.0, cdt)
        agg = lax.dot_general(Rg, m, dn, preferred_element_type=F32)
        csum = lax.dot_general(Rg, cdt, dn, preferred_element_type=F32)
        cnt = jnp.maximum(csum[:, CW - 1:CW], 1.0)
        coord = coord + jnp.where(lane < 3, csum, 0.0) / cnt
        u = _silu(jnp.dot(h, Wn1a[l], preferred_element_type=F32)
                  + jnp.dot(agg, Wn1b[l], preferred_element_type=F32) + bn1)
        h = h + jnp.dot(u, Wn2[l], preferred_element_type=F32) + bn2
    o_ref[...] = jnp.dot(h, Woutr[...], preferred_element_type=F32) + boutr[...]
    co_ref[...] = jnp.where(lane == CW - 1, 1.0, coord)


def _head_body(ins, outs):
    (p1, p2, p3, pc, o4, o5, o6, g1, be1, g2, be2, g3, be3,
     Wfc, bfc, gfc, befc, Wfn, bfn, gfn, befn) = (r[...] for r in ins)
    cnt = jnp.maximum(pc[:, CW - 1:CW], 1.0)

    def bnrelu(x, g, b):
        return jax.nn.relu(x * (g * BNS) + b)

    r1 = bnrelu(p1 / cnt, g1, be1)
    r2 = bnrelu(p2 / cnt, g2, be2)
    r3 = bnrelu(p3 / cnt, g3, be3)
    r4 = bnrelu(o4, g1, be1)
    r5 = bnrelu(o5, g2, be2)
    r6 = bnrelu(o6, g3, be3)
    cat = jnp.concatenate([r1, r4, r2, r5, r3, r6], axis=1)
    x1 = bnrelu(jnp.dot(cat, Wfc, preferred_element_type=F32) + bfc, gfc, befc)
    x2 = bnrelu(jnp.dot(x1, Wfn, preferred_element_type=F32) + bfn, gfn, befn)
    outs[0][...] = jax.nn.sigmoid(x2)


def _sc_gather(tab, gidx):
    nchunks = EPAD // (NWK * CH)
    cpw = nchunks * CH

    @functools.partial(
        pl.kernel,
        out_type=jax.ShapeDtypeStruct((EPAD, GW), F32),
        mesh=_mesh(),
        compiler_params=pltpu.CompilerParams(use_tc_tiling_on_sc=True),
        scratch_types=[pltpu.VMEM((2 * cpw,), jnp.int32),
                       pltpu.VMEM((4, CH, GW), F32)]
        + [pltpu.SemaphoreType.DMA] * 12,
    )
    def gk(tab_hbm, gidx_hbm, out_hbm, idx_v, rows_v, *sems):
        semp, sema, semw = sems[:4], sems[4:8], sems[8:]
        wid = lax.axis_index("s") * 2 + lax.axis_index("c")
        pltpu.sync_copy(gidx_hbm.at[pl.ds(wid * cpw, cpw)],
                        idx_v.at[pl.ds(0, cpw)])
        pltpu.sync_copy(gidx_hbm.at[pl.ds(EPAD + wid * cpw, cpw)],
                        idx_v.at[pl.ds(cpw, cpw)])

        def start_plain(g, b):
            pltpu.async_copy(
                tab_hbm.at[idx_v.at[pl.ds(g * CH, CH)]], rows_v.at[b],
                semp[b])

        def start_add(g, b):
            pltpu.async_copy(
                tab_hbm.at[idx_v.at[pl.ds(cpw + g * CH, CH)]], rows_v.at[b],
                sema[b], add=True)

        def start_write(g, b):
            pltpu.async_copy(
                rows_v.at[b],
                out_hbm.at[pl.ds(wid * cpw + g * CH, CH), :], semw[b])

        def wait_dma(sem, b):
            pltpu.make_async_copy(
                tab_hbm.at[pl.ds(0, CH), :], rows_v.at[b], sem[b]).wait()

        start_plain(0, 0)

        def step(g, b):
            @pl.when(jnp.logical_and(g >= 3, g + 1 < nchunks))
            def _():
                wait_dma(semw, (b + 1) % 4)

            @pl.when(g + 1 < nchunks)
            def _():
                start_plain(g + 1, (b + 1) % 4)

            wait_dma(semp, b)
            start_add(g, b)

            @pl.when(g >= 1)
            def _():
                wait_dma(sema, (b + 3) % 4)
                start_write(g - 1, (b + 3) % 4)

        def body(o, carry):
            for b in range(4):
                step(o * 4 + b, b)
            return carry

        lax.fori_loop(0, nchunks // 4, body, 0)
        wait_dma(sema, (nchunks - 1) % 4)
        start_write(nchunks - 1, (nchunks - 1) % 4)
        for b in range(4):
            wait_dma(semw, b)

    return gk(tab, gidx)


def _sc_scatter(src, ridx, zacc):
    nch = EPAD // (NWK * CH)
    rows_pt = NP_ // 16

    @functools.partial(
        pl.kernel,
        out_type=jax.ShapeDtypeStruct((2, NP_, SW), F32),
        mesh=_mesh(),
        compiler_params=pltpu.CompilerParams(use_tc_tiling_on_sc=True),
        scratch_types=[pltpu.VMEM((nch, CH), jnp.int32),
                       pltpu.VMEM((2, CH, SW), F32),
                       pltpu.VMEM_SHARED((NP_, SW), F32)]
        + [pltpu.SemaphoreType.DMA] * 2,
    )
    def sk(src_hbm, ridx_hbm, z_hbm, out_hbm, idx_v, src_v, acc_sh, *sems):
        semr = sems
        c = lax.axis_index("c")
        s = lax.axis_index("s")
        wid = s * 2 + c
        pltpu.sync_copy(ridx_hbm.at[wid], idx_v)
        pltpu.sync_copy(z_hbm, acc_sh.at[pl.ds(s * rows_pt, rows_pt), :])
        plsc.subcore_barrier()

        def start_load(g, b):
            pltpu.async_copy(
                src_hbm.at[pl.ds((wid * nch + g) * CH, CH), :],
                src_v.at[b], semr[b])

        def wait_load(b):
            pltpu.make_async_copy(
                src_hbm.at[pl.ds(0, CH), :], src_v.at[b], semr[b]).wait()

        for b in range(2):
            start_load(b, b)

        def body(o, carry):
            for b in range(2):
                g = o * 2 + b
                wait_load(b)
                pltpu.sync_copy(src_v.at[b], acc_sh.at[idx_v.at[g]],
                                add=True)

                @pl.when(g + 2 < nch)
                def _():
                    start_load(g + 2, b)
            return carry

        lax.fori_loop(0, nch // 2, body, 0)
        plsc.subcore_barrier()
        pltpu.sync_copy(acc_sh.at[pl.ds(s * rows_pt, rows_pt), :],
                        out_hbm.at[c, pl.ds(s * rows_pt, rows_pt), :])

    return sk(src, ridx, zacc)


def _b(v):
    return v.reshape(1, -1)


def _prep_gcl(p):
    return {
        "W1": p["edge1"]["W"],
        "b1": _b(p["edge1"]["b"]),
        "W2": p["edge2"]["W"], "b2": _b(p["edge2"]["b"]),
        "Watt": p["att"]["W"], "batt": _b(p["att"]["b"]),
        "Wc1": p["coord1"]["W"], "bc1": _b(p["coord1"]["b"]),
        "Wc2": p["coord2W"],
        "Wn1": p["node1"]["W"], "bn1": _b(p["node1"]["b"]),
        "Wn2": p["node2"]["W"], "bn2": _b(p["node2"]["b"]),
    }


def _small_onehots():
    pairs = np.array(list(itertools.combinations(range(BG), 2)),
                     dtype=np.int64).T
    R = np.zeros((EP_SMALL, BG), np.float32)
    C = np.zeros((EP_SMALL, BG), np.float32)
    ne = pairs.shape[1]
    R[np.arange(ne), pairs[0]] = 1.0
    C[np.arange(ne), pairs[1]] = 1.0
    return jnp.asarray(R), jnp.asarray(C)


def _small_params(gp):
    gcls = gp["gcls"]
    W1s, W1bs, W2s, Wc1s, Wn1as, Wn1bs, Wn2s, WVs, AUXs = \
        [], [], [], [], [], [], [], [], []
    for p in gcls:
        W1 = p["edge1"]["W"]
        W1s.append(W1[0:HID])
        W1bs.append(W1[HID:2 * HID])
        W2s.append(p["edge2"]["W"])
        Wc1s.append(p["coord1"]["W"])
        Wn1as.append(p["node1"]["W"][0:HID])
        Wn1bs.append(p["node1"]["W"][HID:2 * HID])
        Wn2s.append(p["node2"]["W"])
        WVs.append(jnp.concatenate([p["att"]["W"], p["coord2W"]], axis=1))
        aux = jnp.stack([
            W1[2 * HID], p["edge1"]["b"], p["edge2"]["b"], p["coord1"]["b"],
            p["node1"]["b"], p["node2"]["b"],
            jnp.full((HID,), p["att"]["b"][0], F32), jnp.zeros((HID,), F32)])
        AUXs.append(aux)
    st = lambda xs: jnp.stack(xs)
    return (st(W1s), st(W1bs), st(W2s), st(Wc1s), st(Wn1as), st(Wn1bs),
            st(Wn2s), st(WVs), st(AUXs),
            gp["emb_in"]["W"], _b(gp["emb_in"]["b"]),
            gp["emb_out"]["W"], _b(gp["emb_out"]["b"]))


def kernel(x_res, x_emb_seq, edge_attr, x_pos, params, edge_index, x_batch):
    row = edge_index[0].astype(jnp.int32)
    col = edge_index[1].astype(jnp.int32)
    npadr = (EPAD - E) // CH
    row2 = row.reshape(E // CH, CH)
    colN2 = (col + N).reshape(E // CH, CH)
    zpad = jnp.zeros((npadr, CH), jnp.int32)
    gidx = jnp.concatenate([row2, zpad, colN2, zpad], 0).reshape(GR)
    ridx = jnp.concatenate(
        [row2, jnp.full((npadr, CH), N, jnp.int32)],
        0).reshape(NWK, EPAD // (NWK * CH), CH)
    ea = jnp.concatenate(
        [edge_attr[:, 0:1].reshape(E // CH, CH),
         jnp.zeros((npadr, CH), F32)], 0).reshape(EPAD, 1)
    coord0 = jnp.pad(x_pos, ((0, 0), (0, CW - 3)))
    zacc = jnp.zeros((NP_ // 16, SW), F32)
    xb = x_batch.astype(jnp.int32).reshape(N, 1)

    p1 = params["egnn1"]
    h = _rows_call(_mm_fn, N, NBLK, [x_res],
                   [p1["emb_in"]["W"], _b(p1["emb_in"]["b"])], [HID])[0]
    coord = coord0
    os_ = []
    for e in (1, 2, 3):
        gp = params[f"egnn{e}"]
        for l in range(2):
            prm = _prep_gcl(gp["gcls"][l])
            tab = _tables_call(h, coord, prm["W1"],
                               prm["b1"]).reshape(2 * N, GW)
            gath = _sc_gather(tab, gidx)
            src = _edge_call(gath, ea, prm)
            parts = _sc_scatter(src, ridx, zacc)
            h, coord = _rows_call(
                _update_fn, N, NBLK,
                [h, coord, parts[0, :N], parts[1, :N]],
                [prm["Wn1"], prm["bn1"], prm["Wn2"], prm["bn2"]],
                [HID, CW])
        consts = [gp["emb_out"]["W"], _b(gp["emb_out"]["b"])]
        widths = [gp["emb_out"]["W"].shape[1]]
        if e < 3:
            nxt = params[f"egnn{e + 1}"]
            consts += [nxt["emb_in"]["W"], _b(nxt["emb_in"]["b"])]
            widths += [HID]
            o, h = _rows_call(_trans_fn, N, NBLK, [h], consts, widths)
        else:
            o, = _rows_call(_trans_fn, N, NBLK, [h], consts, widths)
        os_.append(o)

    pool1, pool2, pool3, poolc = _acc_call(
        _pool_fn, N, NBLK, [os_[0], os_[1], os_[2], coord0, xb], [],
        [(BG, os_[0].shape[1]), (BG, os_[1].shape[1]),
         (BG, os_[2].shape[1]), (BG, CW)])

    Rg, Cg = _small_onehots()
    hs, cs = x_emb_seq, poolc
    small_os = []
    for e in (4, 5, 6):
        (W1a, W1b, W2, Wc1, Wn1a, Wn1b, Wn2, WV, AUX,
         Win, bin_, Wout, bout) = _small_params(params[f"egnn{e}"])
        o, cs = _full_call(
            _small_body,
            [hs, cs, Rg, Cg, Win, bin_, W1a, W1b, W2, Wc1, Wn1a, Wn1b,
             Wn2, WV, AUX, Wout, bout],
            [(BG, Wout.shape[1]), (BG, CW)])
        small_os.append(o)
        hs = o

    fp = params["fc1"]
    fn = params["final"]
    out, = _full_call(
        _head_body,
        [pool1, pool2, pool3, poolc, small_os[0], small_os[1], small_os[2],
         _b(params["bnrelu1"]["gamma"]), _b(params["bnrelu1"]["beta"]),
         _b(params["bnrelu2"]["gamma"]), _b(params["bnrelu2"]["beta"]),
         _b(params["bnrelu3"]["gamma"]), _b(params["bnrelu3"]["beta"]),
         fp["lin"]["W"], _b(fp["lin"]["b"]), _b(fp["gamma"]), _b(fp["beta"]),
         fn["lin"]["W"], _b(fn["lin"]["b"]), _b(fn["gamma"]), _b(fn["beta"])],
        [(BG, fn["lin"]["W"].shape[1])])
    return out


# speedup vs baseline: 1.3881x; 1.1777x over previous
import functools
import itertools

import numpy as np
import jax
import jax.numpy as jnp
from jax import lax
from jax.experimental import pallas as pl
from jax.experimental.pallas import tpu as pltpu
from jax.experimental.pallas import tpu_sc as plsc

N = 10000
E = 320000
BG = 64
HID = 64
CW = 16
GW = 128
SW = 128
EB = 2048
CH = 128
NWK = 32
EPAD = 327680
EPAD2 = EPAD // 2
GR = 2 * EPAD
NP_ = 10112
NBLK = 2000
EP_SMALL = 2048
BNS = float(1.0 / np.sqrt(1.0 + 1e-5))
F32 = jnp.float32

_INTERP = False


def _silu(x):
    return x * jax.nn.sigmoid(x)


def _mesh():
    return plsc.VectorSubcoreMesh(core_axis_name="c", subcore_axis_name="s")


def _rows_call(fn, n_rows, blk, rows, consts, out_widths, out_dtypes=None):
    grid = (n_rows // blk,)
    if out_dtypes is None:
        out_dtypes = [F32] * len(out_widths)
    in_specs = (
        [pl.BlockSpec((blk, r.shape[1]), lambda i: (i, 0)) for r in rows]
        + [pl.BlockSpec(c.shape, lambda i, nd=c.ndim: (0,) * nd) for c in consts]
    )
    out_specs = [pl.BlockSpec((blk, w), lambda i: (i, 0)) for w in out_widths]
    out_shape = [jax.ShapeDtypeStruct((n_rows, w), d)
                 for w, d in zip(out_widths, out_dtypes)]
    nr, nc = len(rows), len(consts)

    def body(*refs):
        fn(refs[:nr], refs[nr:nr + nc], refs[nr + nc:], pl.program_id(0))

    return pl.pallas_call(body, grid=grid, in_specs=in_specs,
                          out_specs=out_specs, out_shape=out_shape,
                          interpret=_INTERP)(*rows, *consts)


def _acc_call(fn, n_rows, blk, rows, consts, out_shapes):
    grid = (n_rows // blk,)
    in_specs = (
        [pl.BlockSpec((blk, r.shape[1]), lambda i: (i, 0)) for r in rows]
        + [pl.BlockSpec(c.shape, lambda i, nd=c.ndim: (0,) * nd) for c in consts]
    )
    out_specs = [pl.BlockSpec(s, lambda i, nd=len(s): (0,) * nd)
                 for s in out_shapes]
    out_shape = [jax.ShapeDtypeStruct(s, F32) for s in out_shapes]
    nr, nc = len(rows), len(consts)

    def body(*refs):
        fn(refs[:nr], refs[nr:nr + nc], refs[nr + nc:], pl.program_id(0))

    return pl.pallas_call(body, grid=grid, in_specs=in_specs,
                          out_specs=out_specs, out_shape=out_shape,
                          interpret=_INTERP)(*rows, *consts)


def _full_call(fn, ins, out_shapes):
    in_specs = [pl.BlockSpec(a.shape, lambda nd=a.ndim: (0,) * nd) for a in ins]
    out_specs = [pl.BlockSpec(s, lambda nd=len(s): (0,) * nd)
                 for s in out_shapes]
    out_shape = [jax.ShapeDtypeStruct(s, F32) for s in out_shapes]
    ni = len(ins)

    def body(*refs):
        fn(refs[:ni], refs[ni:])

    return pl.pallas_call(body, grid=(), in_specs=in_specs,
                          out_specs=out_specs, out_shape=out_shape,
                          interpret=_INTERP)(*ins)


def _mm_fn(ins, consts, outs, i):
    x, = ins
    W, b = (c[...] for c in consts)
    outs[0][...] = jnp.dot(x[...], W, preferred_element_type=F32) + b


def _tables_call(h, coord, W1, b1):
    def body(hr, cdr, W1r, b1r, out):
        k = pl.program_id(0)
        h_ = hr[...]
        cd = cdr[...]
        W = W1r[...]
        z = jnp.zeros((h_.shape[0], GW - HID - CW), F32)

        @pl.when(k == 0)
        def _():
            A = jnp.dot(h_, W[0:HID], preferred_element_type=F32) + b1r[...]
            out[0] = jnp.concatenate([A, cd, z], 1)

        @pl.when(k == 1)
        def _():
            Bv = jnp.dot(h_, W[HID:2 * HID], preferred_element_type=F32)
            out[0] = jnp.concatenate([Bv, -cd, z], 1)

    return pl.pallas_call(
        body, grid=(2, N // NBLK),
        in_specs=[pl.BlockSpec((NBLK, HID), lambda k, i: (i, 0)),
                  pl.BlockSpec((NBLK, CW), lambda k, i: (i, 0)),
                  pl.BlockSpec(W1.shape, lambda k, i: (0, 0)),
                  pl.BlockSpec(b1.shape, lambda k, i: (0, 0))],
        out_specs=pl.BlockSpec((1, NBLK, GW), lambda k, i: (k, i, 0)),
        out_shape=jax.ShapeDtypeStruct((2, N, GW), F32),
        interpret=_INTERP)(h, coord, W1, b1)


def _edge_call(gath, ea, prm):
    epad = gath.shape[0]
    grid = (epad // EB,)
    consts = [prm["W2"], prm["b2"], prm["Watt"], prm["batt"], prm["Wc1"],
              prm["bc1"], prm["Wc2"], prm["W1"]]

    def body(g1r, ear, W2, b2, Watt, batt, Wc1, bc1, Wc2, W1r, out):
        g1 = g1r[...]
        wr = W1r[2 * HID:2 * HID + 1]
        we = W1r[2 * HID + 1:2 * HID + 2]
        a = g1[:, :HID]
        cdd = g1[:, HID:HID + CW]
        radial = jnp.sum(cdd * cdd, axis=1, keepdims=True)
        e1 = _silu(a + radial * wr + ear[...] * we)
        m = _silu(jnp.dot(e1, W2[...], preferred_element_type=F32) + b2[...])
        att = jax.nn.sigmoid(
            jnp.dot(m, Watt[...], preferred_element_type=F32) + batt[...])
        m = m * att
        t = jnp.tanh(jnp.dot(
            _silu(jnp.dot(m, Wc1[...], preferred_element_type=F32) + bc1[...]),
            Wc2[...], preferred_element_type=F32))
        cdt = cdd * (t / (jnp.sqrt(radial) + 1e-8))
        lane = lax.broadcasted_iota(jnp.int32, (EB, CW), 1)
        cdt = jnp.where(lane == CW - 1, 1.0, cdt)
        z = jnp.zeros((EB, SW - HID - CW), F32)
        out[...] = jnp.concatenate([m, cdt, z], axis=1)

    in_specs = (
        [pl.BlockSpec((EB, GW), lambda i: (i, 0)),
         pl.BlockSpec((EB, 1), lambda i: (i, 0))]
        + [pl.BlockSpec(c.shape, lambda i, nd=c.ndim: (0,) * nd)
           for c in consts]
    )
    return pl.pallas_call(
        body, grid=grid, in_specs=in_specs,
        out_specs=pl.BlockSpec((EB, SW), lambda i: (i, 0)),
        out_shape=jax.ShapeDtypeStruct((epad, SW), F32),
        interpret=_INTERP)(gath, ea, *consts)


def _update_fn(ins, consts, outs, i):
    h, cd, p0, p1, p2, p3 = (r[...] for r in ins)
    Wn1, bn1, Wn2, bn2 = (c[...] for c in consts)
    P = (p0 + p1) + (p2 + p3)
    agg = P[:, :HID]
    u = _silu(jnp.dot(h, Wn1[0:HID], preferred_element_type=F32)
              + jnp.dot(agg, Wn1[HID:2 * HID], preferred_element_type=F32)
              + bn1)
    outs[0][...] = h + jnp.dot(u, Wn2, preferred_element_type=F32) + bn2
    rest = P[:, HID:HID + CW]
    cnt = jnp.maximum(rest[:, CW - 1:CW], 1.0)
    lane = lax.broadcasted_iota(jnp.int32, rest.shape, 1)
    outs[1][...] = cd + jnp.where(lane < 3, rest, 0.0) / cnt


def _trans_fn(ins, consts, outs, i):
    h, = ins
    Wo, bo = consts[0][...], consts[1][...]
    o = jnp.dot(h[...], Wo, preferred_element_type=F32) + bo
    outs[0][...] = o
    if len(outs) > 1:
        Wi, bi = consts[2][...], consts[3][...]
        outs[1][...] = jnp.dot(o, Wi, preferred_element_type=F32) + bi


def _pool_fn(ins, consts, outs, i):
    o1, o2, o3, cd, xb = (r[...] for r in ins)
    oh = (xb == lax.broadcasted_iota(jnp.int32, (xb.shape[0], BG), 1)
          ).astype(F32)
    lane = lax.broadcasted_iota(jnp.int32, cd.shape, 1)
    cdc = jnp.where(lane < 3, cd, 0.0)
    cdc = jnp.where(lane == CW - 1, 1.0, cdc)
    dn = (((0,), (0,)), ((), ()))
    for o_ref, dat in zip(outs, (o1, o2, o3, cdc)):
        v = lax.dot_general(oh, dat, dn, preferred_element_type=F32)

        @pl.when(i == 0)
        def _():
            o_ref[...] = v

        @pl.when(i > 0)
        def _():
            o_ref[...] = o_ref[...] + v


def _small_body(ins, outs):
    (h0r, csr, Rr, Cr, Winr, binr, W1a, W1b, W2, Wc1, Wn1a, Wn1b, Wn2, WV,
     AUX, Woutr, boutr) = ins
    o_ref, co_ref = outs
    Rg = Rr[...]
    Cg = Cr[...]
    lane = lax.broadcasted_iota(jnp.int32, (BG, CW), 1)
    elane = lax.broadcasted_iota(jnp.int32, (EP_SMALL, CW), 1)
    cs = csr[...]
    cnt0 = jnp.maximum(cs[:, CW - 1:CW], 1.0)
    coord = jnp.where(lane < 3, cs / cnt0, 0.0)
    h = jnp.dot(h0r[...], Winr[...], preferred_element_type=F32) + binr[...]
    dn = (((0,), (0,)), ((), ()))
    for l in range(2):
        aux = AUX[l]
        wr, b1, b2, bc1, bn1, bn2 = (aux[k:k + 1] for k in range(6))
        batt = aux[6:7, 0:1]
        wv = WV[l]
        hr = jnp.dot(Rg, h, preferred_element_type=F32)
        hc = jnp.dot(Cg, h, preferred_element_type=F32)
        cr = jnp.dot(Rg, coord, preferred_element_type=F32)
        cc = jnp.dot(Cg, coord, preferred_element_type=F32)
        cdd = cr - cc
        radial = jnp.sum(cdd * cdd, axis=1, keepdims=True)
        e1 = _silu(jnp.dot(hr, W1a[l], preferred_element_type=F32)
                   + jnp.dot(hc, W1b[l], preferred_element_type=F32)
                   + radial * wr + b1)
        m = _silu(jnp.dot(e1, W2[l], preferred_element_type=F32) + b2)
        att = jax.nn.sigmoid(
            jnp.dot(m, wv[:, 0:1], preferred_element_type=F32) + batt)
        m = m * att
        t = jnp.tanh(jnp.dot(
            _silu(jnp.dot(m, Wc1[l], preferred_element_type=F32) + bc1),
            wv[:, 1:2], preferred_element_type=F32))
        cdt = cdd * (t / (jnp.sqrt(radial) + 1e-8))
        cdt = jnp.where(elane == CW - 1, 1.0, cdt)
        agg = lax.dot_general(Rg, m, dn, preferred_element_type=F32)
        csum = lax.dot_general(Rg, cdt, dn, preferred_element_type=F32)
        cnt = jnp.maximum(csum[:, CW - 1:CW], 1.0)
        coord = coord + jnp.where(lane < 3, csum, 0.0) / cnt
        u = _silu(jnp.dot(h, Wn1a[l], preferred_element_type=F32)
                  + jnp.dot(agg, Wn1b[l], preferred_element_type=F32) + bn1)
        h = h + jnp.dot(u, Wn2[l], preferred_element_type=F32) + bn2
    o_ref[...] = jnp.dot(h, Woutr[...], preferred_element_type=F32) + boutr[...]
    co_ref[...] = jnp.where(lane == CW - 1, 1.0, coord)


def _head_body(ins, outs):
    (p1, p2, p3, pc, o4, o5, o6, g1, be1, g2, be2, g3, be3,
     Wfc, bfc, gfc, befc, Wfn, bfn, gfn, befn) = (r[...] for r in ins)
    cnt = jnp.maximum(pc[:, CW - 1:CW], 1.0)

    def bnrelu(x, g, b):
        return jax.nn.relu(x * (g * BNS) + b)

    r1 = bnrelu(p1 / cnt, g1, be1)
    r2 = bnrelu(p2 / cnt, g2, be2)
    r3 = bnrelu(p3 / cnt, g3, be3)
    r4 = bnrelu(o4, g1, be1)
    r5 = bnrelu(o5, g2, be2)
    r6 = bnrelu(o6, g3, be3)
    cat = jnp.concatenate([r1, r4, r2, r5, r3, r6], axis=1)
    x1 = bnrelu(jnp.dot(cat, Wfc, preferred_element_type=F32) + bfc, gfc, befc)
    x2 = bnrelu(jnp.dot(x1, Wfn, preferred_element_type=F32) + bfn, gfn, befn)
    outs[0][...] = jax.nn.sigmoid(x2)


def _sc_gather(tab, gidx):
    epad = gidx.shape[0] // 2
    nchunks = epad // (NWK * CH)
    cpw = nchunks * CH

    @functools.partial(
        pl.kernel,
        out_type=jax.ShapeDtypeStruct((epad, GW), F32),
        mesh=_mesh(),
        compiler_params=pltpu.CompilerParams(use_tc_tiling_on_sc=True),
        scratch_types=[pltpu.VMEM((2 * cpw,), jnp.int32),
                       pltpu.VMEM((4, CH, GW), F32)]
        + [pltpu.SemaphoreType.DMA] * 12,
    )
    def gk(tab_hbm, gidx_hbm, out_hbm, idx_v, rows_v, *sems):
        semp, sema, semw = sems[:4], sems[4:8], sems[8:]
        wid = lax.axis_index("s") * 2 + lax.axis_index("c")
        pltpu.sync_copy(gidx_hbm.at[pl.ds(wid * cpw, cpw)],
                        idx_v.at[pl.ds(0, cpw)])
        pltpu.sync_copy(gidx_hbm.at[pl.ds(epad + wid * cpw, cpw)],
                        idx_v.at[pl.ds(cpw, cpw)])

        def start_plain(g, b):
            pltpu.async_copy(
                tab_hbm.at[idx_v.at[pl.ds(g * CH, CH)]], rows_v.at[b],
                semp[b])

        def start_add(g, b):
            pltpu.async_copy(
                tab_hbm.at[idx_v.at[pl.ds(cpw + g * CH, CH)]], rows_v.at[b],
                sema[b], add=True)

        def start_write(g, b):
            pltpu.async_copy(
                rows_v.at[b],
                out_hbm.at[pl.ds(wid * cpw + g * CH, CH), :], semw[b])

        def wait_dma(sem, b):
            pltpu.make_async_copy(
                tab_hbm.at[pl.ds(0, CH), :], rows_v.at[b], sem[b]).wait()

        start_plain(0, 0)

        def step(g, b):
            @pl.when(jnp.logical_and(g >= 3, g + 1 < nchunks))
            def _():
                wait_dma(semw, (b + 1) % 4)

            @pl.when(g + 1 < nchunks)
            def _():
                start_plain(g + 1, (b + 1) % 4)

            wait_dma(semp, b)
            start_add(g, b)

            @pl.when(g >= 1)
            def _():
                wait_dma(sema, (b + 3) % 4)
                start_write(g - 1, (b + 3) % 4)

        def body(o, carry):
            for b in range(4):
                step(o * 4 + b, b)
            return carry

        lax.fori_loop(0, nchunks // 4, body, 0)
        wait_dma(sema, (nchunks - 1) % 4)
        start_write(nchunks - 1, (nchunks - 1) % 4)
        for b in range(4):
            wait_dma(semw, b)

    return gk(tab, gidx)


def _sc_scatter(src, ridx, zacc):
    nch = ridx.shape[1]
    rows_pt = NP_ // 16

    @functools.partial(
        pl.kernel,
        out_type=jax.ShapeDtypeStruct((2, NP_, SW), F32),
        mesh=_mesh(),
        compiler_params=pltpu.CompilerParams(use_tc_tiling_on_sc=True),
        scratch_types=[pltpu.VMEM((nch, CH), jnp.int32),
                       pltpu.VMEM((2, CH, SW), F32),
                       pltpu.VMEM_SHARED((NP_, SW), F32)]
        + [pltpu.SemaphoreType.DMA] * 2,
    )
    def sk(src_hbm, ridx_hbm, z_hbm, out_hbm, idx_v, src_v, acc_sh, *sems):
        semr = sems
        c = lax.axis_index("c")
        s = lax.axis_index("s")
        wid = s * 2 + c
        pltpu.sync_copy(ridx_hbm.at[wid], idx_v)
        pltpu.sync_copy(z_hbm, acc_sh.at[pl.ds(s * rows_pt, rows_pt), :])
        plsc.subcore_barrier()

        def start_load(g, b):
            pltpu.async_copy(
                src_hbm.at[pl.ds((wid * nch + g) * CH, CH), :],
                src_v.at[b], semr[b])

        def wait_load(b):
            pltpu.make_async_copy(
                src_hbm.at[pl.ds(0, CH), :], src_v.at[b], semr[b]).wait()

        for b in range(2):
            start_load(b, b)

        def body(o, carry):
            for b in range(2):
                g = o * 2 + b
                wait_load(b)
                pltpu.sync_copy(src_v.at[b], acc_sh.at[idx_v.at[g]],
                                add=True)

                @pl.when(g + 2 < nch)
                def _():
                    start_load(g + 2, b)
            return carry

        lax.fori_loop(0, nch // 2, body, 0)
        plsc.subcore_barrier()
        pltpu.sync_copy(acc_sh.at[pl.ds(s * rows_pt, rows_pt), :],
                        out_hbm.at[c, pl.ds(s * rows_pt, rows_pt), :])

    return sk(src, ridx, zacc)


def _b(v):
    return v.reshape(1, -1)


def _prep_gcl(p):
    return {
        "W1": p["edge1"]["W"],
        "b1": _b(p["edge1"]["b"]),
        "W2": p["edge2"]["W"], "b2": _b(p["edge2"]["b"]),
        "Watt": p["att"]["W"], "batt": _b(p["att"]["b"]),
        "Wc1": p["coord1"]["W"], "bc1": _b(p["coord1"]["b"]),
        "Wc2": p["coord2W"],
        "Wn1": p["node1"]["W"], "bn1": _b(p["node1"]["b"]),
        "Wn2": p["node2"]["W"], "bn2": _b(p["node2"]["b"]),
    }


def _small_onehots():
    pairs = np.array(list(itertools.combinations(range(BG), 2)),
                     dtype=np.int64).T
    R = np.zeros((EP_SMALL, BG), np.float32)
    C = np.zeros((EP_SMALL, BG), np.float32)
    ne = pairs.shape[1]
    R[np.arange(ne), pairs[0]] = 1.0
    C[np.arange(ne), pairs[1]] = 1.0
    return jnp.asarray(R), jnp.asarray(C)


def _small_params(gp):
    gcls = gp["gcls"]
    W1s, W1bs, W2s, Wc1s, Wn1as, Wn1bs, Wn2s, WVs, AUXs = \
        [], [], [], [], [], [], [], [], []
    for p in gcls:
        W1 = p["edge1"]["W"]
        W1s.append(W1[0:HID])
        W1bs.append(W1[HID:2 * HID])
        W2s.append(p["edge2"]["W"])
        Wc1s.append(p["coord1"]["W"])
        Wn1as.append(p["node1"]["W"][0:HID])
        Wn1bs.append(p["node1"]["W"][HID:2 * HID])
        Wn2s.append(p["node2"]["W"])
        WVs.append(jnp.concatenate([p["att"]["W"], p["coord2W"]], axis=1))
        aux = jnp.stack([
            W1[2 * HID], p["edge1"]["b"], p["edge2"]["b"], p["coord1"]["b"],
            p["node1"]["b"], p["node2"]["b"],
            jnp.full((HID,), p["att"]["b"][0], F32), jnp.zeros((HID,), F32)])
        AUXs.append(aux)
    st = lambda xs: jnp.stack(xs)
    return (st(W1s), st(W1bs), st(W2s), st(Wc1s), st(Wn1as), st(Wn1bs),
            st(Wn2s), st(WVs), st(AUXs),
            gp["emb_in"]["W"], _b(gp["emb_in"]["b"]),
            gp["emb_out"]["W"], _b(gp["emb_out"]["b"]))


def kernel(x_res, x_emb_seq, edge_attr, x_pos, params, edge_index, x_batch):
    row = edge_index[0].astype(jnp.int32)
    col = edge_index[1].astype(jnp.int32)
    npadr = (EPAD - E) // CH
    row2 = row.reshape(E // CH, CH)
    colN2 = (col + N).reshape(E // CH, CH)
    zpad = jnp.zeros((npadr, CH), jnp.int32)
    hc = EPAD2 // CH
    rowp = jnp.concatenate([row2, jnp.full((npadr, CH), N, jnp.int32)], 0)
    colp = jnp.concatenate([colN2, zpad], 0)
    eap = jnp.concatenate(
        [edge_attr[:, 0:1].reshape(E // CH, CH),
         jnp.zeros((npadr, CH), F32)], 0)
    gidxs, ridxs, eas = [], [], []
    for hfi in range(2):
        sel = slice(hfi * hc, (hfi + 1) * hc)
        rh = rowp[sel]
        gidxs.append(jnp.concatenate(
            [jnp.where(rh == N, 0, rh), colp[sel]], 0).reshape(2 * EPAD2))
        ridxs.append(rh.reshape(NWK, EPAD2 // (NWK * CH), CH))
        eas.append(eap[sel].reshape(EPAD2, 1))
    coord0 = jnp.pad(x_pos, ((0, 0), (0, CW - 3)))
    zacc = jnp.zeros((NP_ // 16, SW), F32)
    xb = x_batch.astype(jnp.int32).reshape(N, 1)

    p1 = params["egnn1"]
    h = _rows_call(_mm_fn, N, NBLK, [x_res],
                   [p1["emb_in"]["W"], _b(p1["emb_in"]["b"])], [HID])[0]
    coord = coord0
    os_ = []
    for e in (1, 2, 3):
        gp = params[f"egnn{e}"]
        for l in range(2):
            prm = _prep_gcl(gp["gcls"][l])
            tab = _tables_call(h, coord, prm["W1"],
                               prm["b1"]).reshape(2 * N, GW)
            srcs = [
                _edge_call(_sc_gather(tab, gidxs[k]), eas[k], prm)
                for k in range(2)]
            parts = [_sc_scatter(srcs[k], ridxs[k], zacc) for k in range(2)]
            h, coord = _rows_call(
                _update_fn, N, NBLK,
                [h, coord, parts[0][0, :N], parts[0][1, :N],
                 parts[1][0, :N], parts[1][1, :N]],
                [prm["Wn1"], prm["bn1"], prm["Wn2"], prm["bn2"]],
                [HID, CW])
        consts = [gp["emb_out"]["W"], _b(gp["emb_out"]["b"])]
        widths = [gp["emb_out"]["W"].shape[1]]
        if e < 3:
            nxt = params[f"egnn{e + 1}"]
            consts += [nxt["emb_in"]["W"], _b(nxt["emb_in"]["b"])]
            widths += [HID]
            o, h = _rows_call(_trans_fn, N, NBLK, [h], consts, widths)
        else:
            o, = _rows_call(_trans_fn, N, NBLK, [h], consts, widths)
        os_.append(o)

    pool1, pool2, pool3, poolc = _acc_call(
        _pool_fn, N, NBLK, [os_[0], os_[1], os_[2], coord0, xb], [],
        [(BG, os_[0].shape[1]), (BG, os_[1].shape[1]),
         (BG, os_[2].shape[1]), (BG, CW)])

    Rg, Cg = _small_onehots()
    hs, cs = x_emb_seq, poolc
    small_os = []
    for e in (4, 5, 6):
        (W1a, W1b, W2, Wc1, Wn1a, Wn1b, Wn2, WV, AUX,
         Win, bin_, Wout, bout) = _small_params(params[f"egnn{e}"])
        o, cs = _full_call(
            _small_body,
            [hs, cs, Rg, Cg, Win, bin_, W1a, W1b, W2, Wc1, Wn1a, Wn1b,
             Wn2, WV, AUX, Wout, bout],
            [(BG, Wout.shape[1]), (BG, CW)])
        small_os.append(o)
        hs = o

    fp = params["fc1"]
    fn = params["final"]
    out, = _full_call(
        _head_body,
        [pool1, pool2, pool3, poolc, small_os[0], small_os[1], small_os[2],
         _b(params["bnrelu1"]["gamma"]), _b(params["bnrelu1"]["beta"]),
         _b(params["bnrelu2"]["gamma"]), _b(params["bnrelu2"]["beta"]),
         _b(params["bnrelu3"]["gamma"]), _b(params["bnrelu3"]["beta"]),
         fp["lin"]["W"], _b(fp["lin"]["b"]), _b(fp["gamma"]), _b(fp["beta"]),
         fn["lin"]["W"], _b(fn["lin"]["b"]), _b(fn["gamma"]), _b(fn["beta"])],
        [(BG, fn["lin"]["W"].shape[1])])
    return out


# speedup vs baseline: 1.3881x; 1.0000x over previous
import functools
import itertools

import numpy as np
import jax
import jax.numpy as jnp
from jax import lax
from jax.experimental import pallas as pl
from jax.experimental.pallas import tpu as pltpu
from jax.experimental.pallas import tpu_sc as plsc

N = 10000
E = 320000
BG = 64
HID = 64
CW = 16
GW = 128
SW = 128
EB = 2048
CH = 128
NWK = 32
EPAD = 327680
EPAD2 = EPAD // 2
GR = 2 * EPAD
NP_ = 10112
NBLK = 2000
EP_SMALL = 2048
BNS = float(1.0 / np.sqrt(1.0 + 1e-5))
F32 = jnp.float32


def _silu(x):
    return x * jax.nn.sigmoid(x)


def _mesh():
    return plsc.VectorSubcoreMesh(core_axis_name="c", subcore_axis_name="s")


def _rows_call(fn, n_rows, blk, rows, consts, out_widths, out_dtypes=None):
    grid = (n_rows // blk,)
    if out_dtypes is None:
        out_dtypes = [F32] * len(out_widths)
    in_specs = (
        [pl.BlockSpec((blk, r.shape[1]), lambda i: (i, 0)) for r in rows]
        + [pl.BlockSpec(c.shape, lambda i, nd=c.ndim: (0,) * nd) for c in consts]
    )
    out_specs = [pl.BlockSpec((blk, w), lambda i: (i, 0)) for w in out_widths]
    out_shape = [jax.ShapeDtypeStruct((n_rows, w), d)
                 for w, d in zip(out_widths, out_dtypes)]
    nr, nc = len(rows), len(consts)

    def body(*refs):
        fn(refs[:nr], refs[nr:nr + nc], refs[nr + nc:], pl.program_id(0))

    return pl.pallas_call(body, grid=grid, in_specs=in_specs,
                          out_specs=out_specs, out_shape=out_shape)(*rows, *consts)


def _acc_call(fn, n_rows, blk, rows, consts, out_shapes):
    grid = (n_rows // blk,)
    in_specs = (
        [pl.BlockSpec((blk, r.shape[1]), lambda i: (i, 0)) for r in rows]
        + [pl.BlockSpec(c.shape, lambda i, nd=c.ndim: (0,) * nd) for c in consts]
    )
    out_specs = [pl.BlockSpec(s, lambda i, nd=len(s): (0,) * nd)
                 for s in out_shapes]
    out_shape = [jax.ShapeDtypeStruct(s, F32) for s in out_shapes]
    nr, nc = len(rows), len(consts)

    def body(*refs):
        fn(refs[:nr], refs[nr:nr + nc], refs[nr + nc:], pl.program_id(0))

    return pl.pallas_call(body, grid=grid, in_specs=in_specs,
                          out_specs=out_specs, out_shape=out_shape)(*rows, *consts)


def _full_call(fn, ins, out_shapes):
    in_specs = [pl.BlockSpec(a.shape, lambda nd=a.ndim: (0,) * nd) for a in ins]
    out_specs = [pl.BlockSpec(s, lambda nd=len(s): (0,) * nd)
                 for s in out_shapes]
    out_shape = [jax.ShapeDtypeStruct(s, F32) for s in out_shapes]
    ni = len(ins)

    def body(*refs):
        fn(refs[:ni], refs[ni:])

    return pl.pallas_call(body, grid=(), in_specs=in_specs,
                          out_specs=out_specs, out_shape=out_shape)(*ins)


def _mm_fn(ins, consts, outs, i):
    x, = ins
    W, b = (c[...] for c in consts)
    outs[0][...] = jnp.dot(x[...], W, preferred_element_type=F32) + b


def _tables_call(h, coord, W1, b1):
    def body(hr, cdr, W1r, b1r, out):
        k = pl.program_id(0)
        h_ = hr[...]
        cd = cdr[...]
        W = W1r[...]
        z = jnp.zeros((h_.shape[0], GW - HID - CW), F32)

        @pl.when(k == 0)
        def _():
            A = jnp.dot(h_, W[0:HID], preferred_element_type=F32) + b1r[...]
            out[0] = jnp.concatenate([A, cd, z], 1)

        @pl.when(k == 1)
        def _():
            Bv = jnp.dot(h_, W[HID:2 * HID], preferred_element_type=F32)
            out[0] = jnp.concatenate([Bv, -cd, z], 1)

    return pl.pallas_call(
        body, grid=(2, N // NBLK),
        in_specs=[pl.BlockSpec((NBLK, HID), lambda k, i: (i, 0)),
                  pl.BlockSpec((NBLK, CW), lambda k, i: (i, 0)),
                  pl.BlockSpec(W1.shape, lambda k, i: (0, 0)),
                  pl.BlockSpec(b1.shape, lambda k, i: (0, 0))],
        out_specs=pl.BlockSpec((1, NBLK, GW), lambda k, i: (k, i, 0)),
        out_shape=jax.ShapeDtypeStruct((2, N, GW), F32))(h, coord, W1, b1)


def _edge_call(gath, ea, prm):
    epad = gath.shape[0]
    grid = (epad // EB,)
    consts = [prm["W2"], prm["b2"], prm["Watt"], prm["batt"], prm["Wc1"],
              prm["bc1"], prm["Wc2"], prm["W1"]]

    def body(g1r, ear, W2, b2, Watt, batt, Wc1, bc1, Wc2, W1r, out):
        g1 = g1r[...]
        wr = W1r[2 * HID:2 * HID + 1]
        we = W1r[2 * HID + 1:2 * HID + 2]
        a = g1[:, :HID]
        cdd = g1[:, HID:HID + CW]
        radial = jnp.sum(cdd * cdd, axis=1, keepdims=True)
        e1 = _silu(a + radial * wr + ear[...] * we)
        m = _silu(jnp.dot(e1, W2[...], preferred_element_type=F32) + b2[...])
        att = jax.nn.sigmoid(
            jnp.dot(m, Watt[...], preferred_element_type=F32) + batt[...])
        m = m * att
        t = jnp.tanh(jnp.dot(
            _silu(jnp.dot(m, Wc1[...], preferred_element_type=F32) + bc1[...]),
            Wc2[...], preferred_element_type=F32))
        cdt = cdd * (t / (jnp.sqrt(radial) + 1e-8))
        lane = lax.broadcasted_iota(jnp.int32, (EB, CW), 1)
        cdt = jnp.where(lane == CW - 1, 1.0, cdt)
        z = jnp.zeros((EB, SW - HID - CW), F32)
        out[...] = jnp.concatenate([m, cdt, z], axis=1)

    in_specs = (
        [pl.BlockSpec((EB, GW), lambda i: (i, 0)),
         pl.BlockSpec((EB, 1), lambda i: (i, 0))]
        + [pl.BlockSpec(c.shape, lambda i, nd=c.ndim: (0,) * nd)
           for c in consts]
    )
    return pl.pallas_call(
        body, grid=grid, in_specs=in_specs,
        out_specs=pl.BlockSpec((EB, SW), lambda i: (i, 0)),
        out_shape=jax.ShapeDtypeStruct((epad, SW), F32))(gath, ea, *consts)


def _update_fn(ins, consts, outs, i):
    h, cd, p0, p1, p2, p3 = (r[...] for r in ins)
    Wn1, bn1, Wn2, bn2 = (c[...] for c in consts)
    P = (p0 + p1) + (p2 + p3)
    agg = P[:, :HID]
    u = _silu(jnp.dot(h, Wn1[0:HID], preferred_element_type=F32)
              + jnp.dot(agg, Wn1[HID:2 * HID], preferred_element_type=F32)
              + bn1)
    outs[0][...] = h + jnp.dot(u, Wn2, preferred_element_type=F32) + bn2
    rest = P[:, HID:HID + CW]
    cnt = jnp.maximum(rest[:, CW - 1:CW], 1.0)
    lane = lax.broadcasted_iota(jnp.int32, rest.shape, 1)
    outs[1][...] = cd + jnp.where(lane < 3, rest, 0.0) / cnt


def _trans_fn(ins, consts, outs, i):
    h, = ins
    Wo, bo = consts[0][...], consts[1][...]
    o = jnp.dot(h[...], Wo, preferred_element_type=F32) + bo
    outs[0][...] = o
    if len(outs) > 1:
        Wi, bi = consts[2][...], consts[3][...]
        outs[1][...] = jnp.dot(o, Wi, preferred_element_type=F32) + bi


def _pool_fn(ins, consts, outs, i):
    o1, o2, o3, cd, xb = (r[...] for r in ins)
    oh = (xb == lax.broadcasted_iota(jnp.int32, (xb.shape[0], BG), 1)
          ).astype(F32)
    lane = lax.broadcasted_iota(jnp.int32, cd.shape, 1)
    cdc = jnp.where(lane < 3, cd, 0.0)
    cdc = jnp.where(lane == CW - 1, 1.0, cdc)
    dn = (((0,), (0,)), ((), ()))
    for o_ref, dat in zip(outs, (o1, o2, o3, cdc)):
        v = lax.dot_general(oh, dat, dn, preferred_element_type=F32)

        @pl.when(i == 0)
        def _():
            o_ref[...] = v

        @pl.when(i > 0)
        def _():
            o_ref[...] = o_ref[...] + v


def _small_body(ins, outs):
    (h0r, csr, Rr, Cr, Winr, binr, W1a, W1b, W2, Wc1, Wn1a, Wn1b, Wn2, WV,
     AUX, Woutr, boutr) = ins
    o_ref, co_ref = outs
    Rg = Rr[...]
    Cg = Cr[...]
    lane = lax.broadcasted_iota(jnp.int32, (BG, CW), 1)
    elane = lax.broadcasted_iota(jnp.int32, (EP_SMALL, CW), 1)
    cs = csr[...]
    cnt0 = jnp.maximum(cs[:, CW - 1:CW], 1.0)
    coord = jnp.where(lane < 3, cs / cnt0, 0.0)
    h = jnp.dot(h0r[...], Winr[...], preferred_element_type=F32) + binr[...]
    dn = (((0,), (0,)), ((), ()))
    for l in range(2):
        aux = AUX[l]
        wr, b1, b2, bc1, bn1, bn2 = (aux[k:k + 1] for k in range(6))
        batt = aux[6:7, 0:1]
        wv = WV[l]
        hr = jnp.dot(Rg, h, preferred_element_type=F32)
        hc = jnp.dot(Cg, h, preferred_element_type=F32)
        cr = jnp.dot(Rg, coord, preferred_element_type=F32)
        cc = jnp.dot(Cg, coord, preferred_element_type=F32)
        cdd = cr - cc
        radial = jnp.sum(cdd * cdd, axis=1, keepdims=True)
        e1 = _silu(jnp.dot(hr, W1a[l], preferred_element_type=F32)
                   + jnp.dot(hc, W1b[l], preferred_element_type=F32)
                   + radial * wr + b1)
        m = _silu(jnp.dot(e1, W2[l], preferred_element_type=F32) + b2)
        att = jax.nn.sigmoid(
            jnp.dot(m, wv[:, 0:1], preferred_element_type=F32) + batt)
        m = m * att
        t = jnp.tanh(jnp.dot(
            _silu(jnp.dot(m, Wc1[l], preferred_element_type=F32) + bc1),
            wv[:, 1:2], preferred_element_type=F32))
        cdt = cdd * (t / (jnp.sqrt(radial) + 1e-8))
        cdt = jnp.where(elane == CW - 1, 1.0, cdt)
        agg = lax.dot_general(Rg, m, dn, preferred_element_type=F32)
        csum = lax.dot_general(Rg, cdt, dn, preferred_element_type=F32)
        cnt = jnp.maximum(csum[:, CW - 1:CW], 1.0)
        coord = coord + jnp.where(lane < 3, csum, 0.0) / cnt
        u = _silu(jnp.dot(h, Wn1a[l], preferred_element_type=F32)
                  + jnp.dot(agg, Wn1b[l], preferred_element_type=F32) + bn1)
        h = h + jnp.dot(u, Wn2[l], preferred_element_type=F32) + bn2
    o_ref[...] = jnp.dot(h, Woutr[...], preferred_element_type=F32) + boutr[...]
    co_ref[...] = jnp.where(lane == CW - 1, 1.0, coord)


def _head_body(ins, outs):
    (p1, p2, p3, pc, o4, o5, o6, g1, be1, g2, be2, g3, be3,
     Wfc, bfc, gfc, befc, Wfn, bfn, gfn, befn) = (r[...] for r in ins)
    cnt = jnp.maximum(pc[:, CW - 1:CW], 1.0)

    def bnrelu(x, g, b):
        return jax.nn.relu(x * (g * BNS) + b)

    r1 = bnrelu(p1 / cnt, g1, be1)
    r2 = bnrelu(p2 / cnt, g2, be2)
    r3 = bnrelu(p3 / cnt, g3, be3)
    r4 = bnrelu(o4, g1, be1)
    r5 = bnrelu(o5, g2, be2)
    r6 = bnrelu(o6, g3, be3)
    cat = jnp.concatenate([r1, r4, r2, r5, r3, r6], axis=1)
    x1 = bnrelu(jnp.dot(cat, Wfc, preferred_element_type=F32) + bfc, gfc, befc)
    x2 = bnrelu(jnp.dot(x1, Wfn, preferred_element_type=F32) + bfn, gfn, befn)
    outs[0][...] = jax.nn.sigmoid(x2)


def _sc_gather(tab, gidx):
    epad = gidx.shape[0] // 2
    nchunks = epad // (NWK * CH)
    cpw = nchunks * CH

    @functools.partial(
        pl.kernel,
        out_type=jax.ShapeDtypeStruct((epad, GW), F32),
        mesh=_mesh(),
        compiler_params=pltpu.CompilerParams(use_tc_tiling_on_sc=True),
        scratch_types=[pltpu.VMEM((2 * cpw,), jnp.int32),
                       pltpu.VMEM((4, CH, GW), F32)]
        + [pltpu.SemaphoreType.DMA] * 12,
    )
    def gk(tab_hbm, gidx_hbm, out_hbm, idx_v, rows_v, *sems):
        semp, sema, semw = sems[:4], sems[4:8], sems[8:]
        wid = lax.axis_index("s") * 2 + lax.axis_index("c")
        pltpu.sync_copy(gidx_hbm.at[pl.ds(wid * cpw, cpw)],
                        idx_v.at[pl.ds(0, cpw)])
        pltpu.sync_copy(gidx_hbm.at[pl.ds(epad + wid * cpw, cpw)],
                        idx_v.at[pl.ds(cpw, cpw)])

        def start_plain(g, b):
            pltpu.async_copy(
                tab_hbm.at[idx_v.at[pl.ds(g * CH, CH)]], rows_v.at[b],
                semp[b])

        def start_add(g, b):
            pltpu.async_copy(
                tab_hbm.at[idx_v.at[pl.ds(cpw + g * CH, CH)]], rows_v.at[b],
                sema[b], add=True)

        def start_write(g, b):
            pltpu.async_copy(
                rows_v.at[b],
                out_hbm.at[pl.ds(wid * cpw + g * CH, CH), :], semw[b])

        def wait_dma(sem, b):
            pltpu.make_async_copy(
                tab_hbm.at[pl.ds(0, CH), :], rows_v.at[b], sem[b]).wait()

        start_plain(0, 0)

        def step(g, b):
            @pl.when(jnp.logical_and(g >= 3, g + 1 < nchunks))
            def _():
                wait_dma(semw, (b + 1) % 4)

            @pl.when(g + 1 < nchunks)
            def _():
                start_plain(g + 1, (b + 1) % 4)

            wait_dma(semp, b)
            start_add(g, b)

            @pl.when(g >= 1)
            def _():
                wait_dma(sema, (b + 3) % 4)
                start_write(g - 1, (b + 3) % 4)

        def body(o, carry):
            for b in range(4):
                step(o * 4 + b, b)
            return carry

        lax.fori_loop(0, nchunks // 4, body, 0)
        wait_dma(sema, (nchunks - 1) % 4)
        start_write(nchunks - 1, (nchunks - 1) % 4)
        for b in range(4):
            wait_dma(semw, b)

    return gk(tab, gidx)


def _sc_scatter(src, ridx, zacc):
    nch = ridx.shape[1]
    rows_pt = NP_ // 16

    @functools.partial(
        pl.kernel,
        out_type=jax.ShapeDtypeStruct((2, NP_, SW), F32),
        mesh=_mesh(),
        compiler_params=pltpu.CompilerParams(use_tc_tiling_on_sc=True),
        scratch_types=[pltpu.VMEM((nch, CH), jnp.int32),
                       pltpu.VMEM((2, CH, SW), F32),
                       pltpu.VMEM_SHARED((NP_, SW), F32)]
        + [pltpu.SemaphoreType.DMA] * 2,
    )
    def sk(src_hbm, ridx_hbm, z_hbm, out_hbm, idx_v, src_v, acc_sh, *sems):
        semr = sems
        c = lax.axis_index("c")
        s = lax.axis_index("s")
        wid = s * 2 + c
        pltpu.sync_copy(ridx_hbm.at[wid], idx_v)
        pltpu.sync_copy(z_hbm, acc_sh.at[pl.ds(s * rows_pt, rows_pt), :])
        plsc.subcore_barrier()

        def start_load(g, b):
            pltpu.async_copy(
                src_hbm.at[pl.ds((wid * nch + g) * CH, CH), :],
                src_v.at[b], semr[b])

        def wait_load(b):
            pltpu.make_async_copy(
                src_hbm.at[pl.ds(0, CH), :], src_v.at[b], semr[b]).wait()

        for b in range(2):
            start_load(b, b)

        def body(o, carry):
            for b in range(2):
                g = o * 2 + b
                wait_load(b)
                pltpu.sync_copy(src_v.at[b], acc_sh.at[idx_v.at[g]],
                                add=True)

                @pl.when(g + 2 < nch)
                def _():
                    start_load(g + 2, b)
            return carry

        lax.fori_loop(0, nch // 2, body, 0)
        plsc.subcore_barrier()
        pltpu.sync_copy(acc_sh.at[pl.ds(s * rows_pt, rows_pt), :],
                        out_hbm.at[c, pl.ds(s * rows_pt, rows_pt), :])

    return sk(src, ridx, zacc)


def _b(v):
    return v.reshape(1, -1)


def _prep_gcl(p):
    return {
        "W1": p["edge1"]["W"],
        "b1": _b(p["edge1"]["b"]),
        "W2": p["edge2"]["W"], "b2": _b(p["edge2"]["b"]),
        "Watt": p["att"]["W"], "batt": _b(p["att"]["b"]),
        "Wc1": p["coord1"]["W"], "bc1": _b(p["coord1"]["b"]),
        "Wc2": p["coord2W"],
        "Wn1": p["node1"]["W"], "bn1": _b(p["node1"]["b"]),
        "Wn2": p["node2"]["W"], "bn2": _b(p["node2"]["b"]),
    }


def _small_onehots():
    pairs = np.array(list(itertools.combinations(range(BG), 2)),
                     dtype=np.int64).T
    R = np.zeros((EP_SMALL, BG), np.float32)
    C = np.zeros((EP_SMALL, BG), np.float32)
    ne = pairs.shape[1]
    R[np.arange(ne), pairs[0]] = 1.0
    C[np.arange(ne), pairs[1]] = 1.0
    return jnp.asarray(R), jnp.asarray(C)


def _small_params(gp):
    gcls = gp["gcls"]
    W1s, W1bs, W2s, Wc1s, Wn1as, Wn1bs, Wn2s, WVs, AUXs = \
        [], [], [], [], [], [], [], [], []
    for p in gcls:
        W1 = p["edge1"]["W"]
        W1s.append(W1[0:HID])
        W1bs.append(W1[HID:2 * HID])
        W2s.append(p["edge2"]["W"])
        Wc1s.append(p["coord1"]["W"])
        Wn1as.append(p["node1"]["W"][0:HID])
        Wn1bs.append(p["node1"]["W"][HID:2 * HID])
        Wn2s.append(p["node2"]["W"])
        WVs.append(jnp.concatenate([p["att"]["W"], p["coord2W"]], axis=1))
        aux = jnp.stack([
            W1[2 * HID], p["edge1"]["b"], p["edge2"]["b"], p["coord1"]["b"],
            p["node1"]["b"], p["node2"]["b"],
            jnp.full((HID,), p["att"]["b"][0], F32), jnp.zeros((HID,), F32)])
        AUXs.append(aux)
    st = lambda xs: jnp.stack(xs)
    return (st(W1s), st(W1bs), st(W2s), st(Wc1s), st(Wn1as), st(Wn1bs),
            st(Wn2s), st(WVs), st(AUXs),
            gp["emb_in"]["W"], _b(gp["emb_in"]["b"]),
            gp["emb_out"]["W"], _b(gp["emb_out"]["b"]))


def kernel(x_res, x_emb_seq, edge_attr, x_pos, params, edge_index, x_batch):
    row = edge_index[0].astype(jnp.int32)
    col = edge_index[1].astype(jnp.int32)
    npadr = (EPAD - E) // CH
    row2 = row.reshape(E // CH, CH)
    colN2 = (col + N).reshape(E // CH, CH)
    zpad = jnp.zeros((npadr, CH), jnp.int32)
    hc = EPAD2 // CH
    rowp = jnp.concatenate([row2, jnp.full((npadr, CH), N, jnp.int32)], 0)
    colp = jnp.concatenate([colN2, zpad], 0)
    eap = jnp.concatenate(
        [edge_attr[:, 0:1].reshape(E // CH, CH),
         jnp.zeros((npadr, CH), F32)], 0)
    gidxs, ridxs, eas = [], [], []
    for hfi in range(2):
        sel = slice(hfi * hc, (hfi + 1) * hc)
        rh = rowp[sel]
        gidxs.append(jnp.concatenate(
            [jnp.where(rh == N, 0, rh), colp[sel]], 0).reshape(2 * EPAD2))
        ridxs.append(rh.reshape(NWK, EPAD2 // (NWK * CH), CH))
        eas.append(eap[sel].reshape(EPAD2, 1))
    coord0 = jnp.pad(x_pos, ((0, 0), (0, CW - 3)))
    zacc = jnp.zeros((NP_ // 16, SW), F32)
    xb = x_batch.astype(jnp.int32).reshape(N, 1)

    p1 = params["egnn1"]
    h = _rows_call(_mm_fn, N, NBLK, [x_res],
                   [p1["emb_in"]["W"], _b(p1["emb_in"]["b"])], [HID])[0]
    coord = coord0
    os_ = []
    for e in (1, 2, 3):
        gp = params[f"egnn{e}"]
        for l in range(2):
            prm = _prep_gcl(gp["gcls"][l])
            tab = _tables_call(h, coord, prm["W1"],
                               prm["b1"]).reshape(2 * N, GW)
            srcs = [
                _edge_call(_sc_gather(tab, gidxs[k]), eas[k], prm)
                for k in range(2)]
            parts = [_sc_scatter(srcs[k], ridxs[k], zacc) for k in range(2)]
            h, coord = _rows_call(
                _update_fn, N, NBLK,
                [h, coord, parts[0][0, :N], parts[0][1, :N],
                 parts[1][0, :N], parts[1][1, :N]],
                [prm["Wn1"], prm["bn1"], prm["Wn2"], prm["bn2"]],
                [HID, CW])
        consts = [gp["emb_out"]["W"], _b(gp["emb_out"]["b"])]
        widths = [gp["emb_out"]["W"].shape[1]]
        if e < 3:
            nxt = params[f"egnn{e + 1}"]
            consts += [nxt["emb_in"]["W"], _b(nxt["emb_in"]["b"])]
            widths += [HID]
            o, h = _rows_call(_trans_fn, N, NBLK, [h], consts, widths)
        else:
            o, = _rows_call(_trans_fn, N, NBLK, [h], consts, widths)
        os_.append(o)

    pool1, pool2, pool3, poolc = _acc_call(
        _pool_fn, N, NBLK, [os_[0], os_[1], os_[2], coord0, xb], [],
        [(BG, os_[0].shape[1]), (BG, os_[1].shape[1]),
         (BG, os_[2].shape[1]), (BG, CW)])

    Rg, Cg = _small_onehots()
    hs, cs = x_emb_seq, poolc
    small_os = []
    for e in (4, 5, 6):
        (W1a, W1b, W2, Wc1, Wn1a, Wn1b, Wn2, WV, AUX,
         Win, bin_, Wout, bout) = _small_params(params[f"egnn{e}"])
        o, cs = _full_call(
            _small_body,
            [hs, cs, Rg, Cg, Win, bin_, W1a, W1b, W2, Wc1, Wn1a, Wn1b,
             Wn2, WV, AUX, Wout, bout],
            [(BG, Wout.shape[1]), (BG, CW)])
        small_os.append(o)
        hs = o

    fp = params["fc1"]
    fn = params["final"]
    out, = _full_call(
        _head_body,
        [pool1, pool2, pool3, poolc, small_os[0], small_os[1], small_os[2],
         _b(params["bnrelu1"]["gamma"]), _b(params["bnrelu1"]["beta"]),
         _b(params["bnrelu2"]["gamma"]), _b(params["bnrelu2"]["beta"]),
         _b(params["bnrelu3"]["gamma"]), _b(params["bnrelu3"]["beta"]),
         fp["lin"]["W"], _b(fp["lin"]["b"]), _b(fp["gamma"]), _b(fp["beta"]),
         fn["lin"]["W"], _b(fn["lin"]["b"]), _b(fn["gamma"]), _b(fn["beta"])],
        [(BG, fn["lin"]["W"].shape[1])])
    return out
